# Initial kernel scaffold; baseline (speedup 1.0000x reference)
import jax
import jax.numpy as jnp
from jax.experimental import pallas as pl


def kernel(xyz, motion_feature, super_gaussians, super_gaussians_feature):
    raise NotImplementedError("write your pallas kernel here")


# speedup vs baseline: 34.6427x; 34.6427x over previous
import jax
import jax.numpy as jnp
from jax.experimental import pallas as pl

NEAREST = 8
FEATURE_AMPLIFY = 10.0


def _knn_kernel(q_ref, k_ref, dist_ref, idx_ref):
    q = q_ref[...]
    k = k_ref[...]

    q_sq = jnp.sum(q * q, axis=1, keepdims=True)
    k_sq = jnp.sum(k * k, axis=1)

    dims = (((1,), (1,)), ((), ()))
    dot = jax.lax.dot_general(q, k, dims, preferred_element_type=jnp.float32)

    d2 = q_sq + k_sq[None, :] - 2.0 * dot
    d2 = jnp.maximum(d2, 1e-12)

    m = d2.shape[1]
    iota = jax.lax.broadcasted_iota(jnp.int32, d2.shape, 1)
    dists = []
    idxs = []
    for _ in range(NEAREST):
        vmin = jnp.min(d2, axis=1, keepdims=True)
        imin = jnp.min(jnp.where(d2 == vmin, iota, m), axis=1,
                       keepdims=True)
        dists.append(vmin)
        idxs.append(imin)
        d2 = jnp.where(iota == imin, jnp.float32(jnp.inf), d2)

    dist_ref[...] = jnp.sqrt(jnp.concatenate(dists, axis=1))
    idx_ref[...] = jnp.concatenate(idxs, axis=1)


def kernel(xyz, motion_feature, super_gaussians, super_gaussians_feature):
    q = jnp.concatenate([xyz, motion_feature * FEATURE_AMPLIFY], axis=-1)
    k = jnp.concatenate(
        [super_gaussians, super_gaussians_feature * FEATURE_AMPLIFY], axis=-1)
    n, d = q.shape
    m = k.shape[0]
    br = 512
    grid = (pl.cdiv(n, br),)
    dist, idx = pl.pallas_call(
        _knn_kernel,
        grid=grid,
        in_specs=[
            pl.BlockSpec((br, d), lambda i: (i, 0)),
            pl.BlockSpec((m, d), lambda i: (0, 0)),
        ],
        out_specs=[
            pl.BlockSpec((br, NEAREST), lambda i: (i, 0)),
            pl.BlockSpec((br, NEAREST), lambda i: (i, 0)),
        ],
        out_shape=[
            jax.ShapeDtypeStruct((n, NEAREST), jnp.float32),
            jax.ShapeDtypeStruct((n, NEAREST), jnp.int32),
        ],
    )(q, k)
    return dist, idx


# speedup vs baseline: 43.0769x; 1.2435x over previous
import jax
import jax.numpy as jnp
from jax.experimental import pallas as pl
from jax.experimental.pallas import tpu as pltpu

NEAREST = 8
FEATURE_AMPLIFY = 10.0


def _knn_kernel(q_ref, k_ref, dist_ref, idx_ref):
    q = q_ref[...]
    k = k_ref[...]

    q_sq = jnp.sum(q * q, axis=1, keepdims=True)
    k_sq = jnp.sum(k * k, axis=1)

    dims = (((1,), (1,)), ((), ()))
    dot = jax.lax.dot_general(q, k, dims, preferred_element_type=jnp.float32)

    d2 = q_sq + k_sq[None, :] - 2.0 * dot
    d2 = jnp.maximum(d2, 1e-12)

    m = d2.shape[1]
    iota = jax.lax.broadcasted_iota(jnp.int32, d2.shape, 1).astype(jnp.float32)
    dists = []
    idxs = []
    for _ in range(NEAREST):
        vmin = jnp.min(d2, axis=1, keepdims=True)
        imin = jnp.min(jnp.where(d2 == vmin, iota, jnp.float32(m)),
                       axis=1, keepdims=True)
        dists.append(vmin)
        idxs.append(imin)
        d2 = jnp.where(iota == imin, jnp.float32(jnp.inf), d2)

    dist_ref[...] = jnp.sqrt(jnp.concatenate(dists, axis=1))
    idx_ref[...] = jnp.concatenate(idxs, axis=1).astype(jnp.int32)


def kernel(xyz, motion_feature, super_gaussians, super_gaussians_feature):
    q = jnp.concatenate([xyz, motion_feature * FEATURE_AMPLIFY], axis=-1)
    k = jnp.concatenate(
        [super_gaussians, super_gaussians_feature * FEATURE_AMPLIFY], axis=-1)
    n, d = q.shape
    m = k.shape[0]
    br = 512
    grid = (pl.cdiv(n, br),)
    dist, idx = pl.pallas_call(
        _knn_kernel,
        grid=grid,
        in_specs=[
            pl.BlockSpec((br, d), lambda i: (i, 0)),
            pl.BlockSpec((m, d), lambda i: (0, 0)),
        ],
        out_specs=[
            pl.BlockSpec((br, NEAREST), lambda i: (i, 0)),
            pl.BlockSpec((br, NEAREST), lambda i: (i, 0)),
        ],
        out_shape=[
            jax.ShapeDtypeStruct((n, NEAREST), jnp.float32),
            jax.ShapeDtypeStruct((n, NEAREST), jnp.int32),
        ],
        compiler_params=pltpu.CompilerParams(
            dimension_semantics=("parallel",)),
    )(q, k)
    return dist, idx


# speedup vs baseline: 43.7662x; 1.0160x over previous
import jax
import jax.numpy as jnp
from jax.experimental import pallas as pl
from jax.experimental.pallas import tpu as pltpu

NEAREST = 8
FEATURE_AMPLIFY = 10.0


def _knn_kernel(q_ref, k_ref, dist_ref, idx_ref):
    q = q_ref[...]
    k = k_ref[...]

    q_sq = jnp.sum(q * q, axis=1, keepdims=True)
    k_sq = jnp.sum(k * k, axis=1)

    dims = (((1,), (1,)), ((), ()))
    dot = jax.lax.dot_general(q, k, dims, preferred_element_type=jnp.float32)

    d2 = q_sq + k_sq[None, :] - 2.0 * dot
    d2 = jnp.maximum(d2, 1e-12)

    m = d2.shape[1]
    iota = jax.lax.broadcasted_iota(jnp.int32, d2.shape, 1).astype(jnp.float32)
    dists = []
    idxs = []
    for _ in range(NEAREST):
        vmin = jnp.min(d2, axis=1, keepdims=True)
        imin = jnp.min(jnp.where(d2 == vmin, iota, jnp.float32(m)),
                       axis=1, keepdims=True)
        dists.append(vmin)
        idxs.append(imin)
        d2 = jnp.where(iota == imin, jnp.float32(jnp.inf), d2)

    dist_ref[...] = jnp.sqrt(jnp.concatenate(dists, axis=1))
    idx_ref[...] = jnp.concatenate(idxs, axis=1).astype(jnp.int32)


def kernel(xyz, motion_feature, super_gaussians, super_gaussians_feature):
    q = jnp.concatenate([xyz, motion_feature * FEATURE_AMPLIFY], axis=-1)
    k = jnp.concatenate(
        [super_gaussians, super_gaussians_feature * FEATURE_AMPLIFY], axis=-1)
    n, d = q.shape
    m = k.shape[0]
    br = 1024
    grid = (pl.cdiv(n, br),)
    dist, idx = pl.pallas_call(
        _knn_kernel,
        grid=grid,
        in_specs=[
            pl.BlockSpec((br, d), lambda i: (i, 0)),
            pl.BlockSpec((m, d), lambda i: (0, 0)),
        ],
        out_specs=[
            pl.BlockSpec((br, NEAREST), lambda i: (i, 0)),
            pl.BlockSpec((br, NEAREST), lambda i: (i, 0)),
        ],
        out_shape=[
            jax.ShapeDtypeStruct((n, NEAREST), jnp.float32),
            jax.ShapeDtypeStruct((n, NEAREST), jnp.int32),
        ],
        compiler_params=pltpu.CompilerParams(
            dimension_semantics=("parallel",)),
    )(q, k)
    return dist, idx


# speedup vs baseline: 46.7079x; 1.0672x over previous
import jax
import jax.numpy as jnp
from jax.experimental import pallas as pl
from jax.experimental.pallas import tpu as pltpu

NEAREST = 8
FEATURE_AMPLIFY = 10.0


def _knn_kernel(xyz_ref, mf_ref, sg_ref, sgf_ref, dist_ref, idx_ref,
                q_scr, k_scr):
    q_scr[:, 0:3] = xyz_ref[...]
    q_scr[:, 3:131] = mf_ref[...] * FEATURE_AMPLIFY
    k_scr[:, 0:3] = sg_ref[...]
    k_scr[:, 3:131] = sgf_ref[...] * FEATURE_AMPLIFY

    q = q_scr[...]
    k = k_scr[...]

    q_sq = jnp.sum(q * q, axis=1, keepdims=True)
    k_sq = jnp.sum(k * k, axis=1)

    dims = (((1,), (1,)), ((), ()))
    dot = jax.lax.dot_general(q, k, dims, preferred_element_type=jnp.float32)

    d2 = q_sq + k_sq[None, :] - 2.0 * dot
    d2 = jnp.maximum(d2, 1e-12)

    m = d2.shape[1]
    iota = jax.lax.broadcasted_iota(jnp.int32, d2.shape, 1).astype(jnp.float32)
    dists = []
    idxs = []
    for _ in range(NEAREST):
        vmin = jnp.min(d2, axis=1, keepdims=True)
        imin = jnp.min(jnp.where(d2 == vmin, iota, jnp.float32(m)),
                       axis=1, keepdims=True)
        dists.append(vmin)
        idxs.append(imin)
        d2 = jnp.where(iota == imin, jnp.float32(jnp.inf), d2)

    dist_ref[...] = jnp.sqrt(jnp.concatenate(dists, axis=1))
    idx_ref[...] = jnp.concatenate(idxs, axis=1).astype(jnp.int32)


def kernel(xyz, motion_feature, super_gaussians, super_gaussians_feature):
    n = xyz.shape[0]
    m = super_gaussians.shape[0]
    d = 131
    br = 1024
    grid = (pl.cdiv(n, br),)
    dist, idx = pl.pallas_call(
        _knn_kernel,
        grid=grid,
        in_specs=[
            pl.BlockSpec((br, 3), lambda i: (i, 0)),
            pl.BlockSpec((br, 128), lambda i: (i, 0)),
            pl.BlockSpec((m, 3), lambda i: (0, 0)),
            pl.BlockSpec((m, 128), lambda i: (0, 0)),
        ],
        out_specs=[
            pl.BlockSpec((br, NEAREST), lambda i: (i, 0)),
            pl.BlockSpec((br, NEAREST), lambda i: (i, 0)),
        ],
        out_shape=[
            jax.ShapeDtypeStruct((n, NEAREST), jnp.float32),
            jax.ShapeDtypeStruct((n, NEAREST), jnp.int32),
        ],
        scratch_shapes=[
            pltpu.VMEM((br, d), jnp.float32),
            pltpu.VMEM((m, d), jnp.float32),
        ],
        compiler_params=pltpu.CompilerParams(
            dimension_semantics=("parallel",)),
    )(xyz, motion_feature, super_gaussians, super_gaussians_feature)
    return dist, idx
